# baseline (speedup 1.0000x reference)
import jax
import jax.numpy as jnp
from jax.experimental import pallas as pl
from jax.experimental.pallas import tpu as pltpu

HID = 256
N = 4096
BM = 1024
NB = N // BM


def _mix(cs_ref, av_ref, es_ref, z_ref, pair, i):
    c0, c1 = 2 * pair, 2 * pair + 1
    av = av_ref[pair, 0, :]
    l0 = jnp.sum(av * cs_ref[c0, 0, :]) * (1.0 / N)
    l1 = jnp.sum(av * cs_ref[c1, 0, :]) * (1.0 / N)
    m = jnp.maximum(l0, l1)
    x0 = jnp.exp(l0 - m)
    x1 = jnp.exp(l1 - m)
    b0 = x0 / (x0 + x1)
    b1 = x1 / (x0 + x1)
    z_ref[...] = (b0 * es_ref[c0, i].astype(jnp.float32)
                  + b1 * es_ref[c1, i].astype(jnp.float32))


def _body(h_ref, adj0_ref, adj1_ref, adj2_ref, adj3_ref, wt_ref, gp_ref,
          awt_ref, ab_ref, av_ref, z_ref, sf_ref, es_ref, cs_ref):
    k = pl.program_id(0)
    i = pl.program_id(1)

    @pl.when(k < 4)
    def _():
        @pl.when(i == 0)
        def _():
            sf_ref[...] = (
                jnp.dot(h_ref[...].astype(jnp.bfloat16),
                        wt_ref[0].astype(jnp.bfloat16),
                        preferred_element_type=jnp.float32)
                + gp_ref[0, 0, :][None, :]).astype(jnp.bfloat16)

        bias = gp_ref[0, 1, :][None, :]
        a = gp_ref[0, 2, :][None, :]
        awtb = awt_ref[0].astype(jnp.bfloat16)
        abr = ab_ref[k // 2, 0, :][None, :]
        bq = BM // 4
        t = None
        for q, adj_ref in enumerate((adj0_ref, adj1_ref, adj2_ref, adj3_ref)):
            o = jnp.dot(adj_ref[0].astype(jnp.bfloat16), sf_ref[...],
                        preferred_element_type=jnp.float32) + bias
            e = jnp.where(o > 0, o, a * o).astype(jnp.bfloat16)
            es_ref[k, i, q * bq:(q + 1) * bq] = e
            tq = jnp.tanh(
                jnp.dot(e, awtb, preferred_element_type=jnp.float32) + abr)
            t = tq if t is None else t + tq
        part = jnp.sum(t, axis=0, keepdims=True)

        @pl.when(i == 0)
        def _():
            cs_ref[k] = part

        @pl.when(i > 0)
        def _():
            cs_ref[k] = cs_ref[k] + part

    @pl.when(k == 2)
    def _():
        _mix(cs_ref, av_ref, es_ref, z_ref, 0, i)

    @pl.when(k == 4)
    def _():
        _mix(cs_ref, av_ref, es_ref, z_ref, 1, i)


def kernel(h, mps, mp_edge, gcn_W, gcn_bfc, gcn_bias, gcn_a, att_W, att_b,
           att_v):
    del mp_edge
    gwt = jnp.swapaxes(gcn_W, 1, 2)
    awt = jnp.swapaxes(att_W, 1, 2)
    gp = jnp.stack(
        [gcn_bfc, gcn_bias, jnp.broadcast_to(gcn_a[:, None], (4, HID))],
        axis=1)
    ab = att_b[:, None, :]
    av = att_v[:, None, :]

    def zmap(k, i):
        blk = jnp.where(k < 2, 0,
                        jnp.where(k == 2, i,
                                  jnp.where(k == 3, NB - 1, NB + i)))
        return (blk, 0)

    z = pl.pallas_call(
        _body,
        grid=(5, NB),
        in_specs=[
            pl.BlockSpec((N, HID), lambda k, i: (jnp.minimum(k, 3) // 2, 0)),
        ] + [
            pl.BlockSpec(
                (1, BM // 4, N),
                lambda k, i, q=q: (jnp.minimum(k, 3),
                                   4 * jnp.where(k < 4, i, NB - 1) + q, 0))
            for q in range(4)
        ] + [
            pl.BlockSpec((1, HID, HID), lambda k, i: (jnp.minimum(k, 3), 0, 0)),
            pl.BlockSpec((1, 3, HID), lambda k, i: (jnp.minimum(k, 3), 0, 0)),
            pl.BlockSpec((1, HID, HID),
                         lambda k, i: (jnp.minimum(k, 3) // 2, 0, 0)),
            pl.BlockSpec((2, 1, HID), lambda k, i: (0, 0, 0)),
            pl.BlockSpec((2, 1, HID), lambda k, i: (0, 0, 0)),
        ],
        out_specs=pl.BlockSpec((BM, HID), zmap),
        out_shape=jax.ShapeDtypeStruct((2 * N, HID), jnp.float32),
        scratch_shapes=[
            pltpu.VMEM((N, HID), jnp.bfloat16),
            pltpu.VMEM((4, NB, BM, HID), jnp.bfloat16),
            pltpu.VMEM((4, 1, HID), jnp.float32),
        ],
    )(h, mps, mps, mps, mps, gwt, gp, awt, ab, av)
    return z


# speedup vs baseline: 1.0859x; 1.0859x over previous
import jax
import jax.numpy as jnp
from jax.experimental import pallas as pl
from jax.experimental.pallas import tpu as pltpu

HID = 256
N = 4096
BM = 1024
NB = N // BM


def _body(h_ref, adjl_ref, adjr_ref, wt_ref, gp_ref, awt_ref, ab_ref, av_ref,
          z_ref, sf_ref, es_ref, cs_ref):
    c = pl.program_id(1)
    i = pl.program_id(2)

    @pl.when(c < 2)
    def _():
        @pl.when(i == 0)
        def _():
            sf_ref[...] = (
                jnp.dot(h_ref[...].astype(jnp.bfloat16),
                        wt_ref[0].astype(jnp.bfloat16),
                        preferred_element_type=jnp.float32)
                + gp_ref[0, 0, :][None, :]).astype(jnp.bfloat16)

        bias = gp_ref[0, 1, :][None, :]
        a = gp_ref[0, 2, :][None, :]
        ot = jnp.dot(adjl_ref[0].astype(jnp.bfloat16), sf_ref[...],
                     preferred_element_type=jnp.float32) + bias
        ob = jnp.dot(adjr_ref[0].astype(jnp.bfloat16), sf_ref[...],
                     preferred_element_type=jnp.float32) + bias
        et = jnp.where(ot > 0, ot, a * ot).astype(jnp.bfloat16)
        eb = jnp.where(ob > 0, ob, a * ob).astype(jnp.bfloat16)
        es_ref[c, i, :BM // 2] = et
        es_ref[c, i, BM // 2:] = eb
        awtb = awt_ref[0].astype(jnp.bfloat16)
        abr = ab_ref[0, 0, :][None, :]
        t = (jnp.tanh(jnp.dot(et, awtb, preferred_element_type=jnp.float32)
                      + abr)
             + jnp.tanh(jnp.dot(eb, awtb, preferred_element_type=jnp.float32)
                        + abr))
        part = jnp.sum(t, axis=0, keepdims=True)

        @pl.when(i == 0)
        def _():
            cs_ref[c] = part

        @pl.when(i > 0)
        def _():
            cs_ref[c] = cs_ref[c] + part

    @pl.when(c == 2)
    def _():
        av = av_ref[0, 0, :]
        l0 = jnp.sum(av * cs_ref[0, 0, :]) * (1.0 / N)
        l1 = jnp.sum(av * cs_ref[1, 0, :]) * (1.0 / N)
        m = jnp.maximum(l0, l1)
        x0 = jnp.exp(l0 - m)
        x1 = jnp.exp(l1 - m)
        b0 = x0 / (x0 + x1)
        b1 = x1 / (x0 + x1)
        z_ref[...] = (b0 * es_ref[0, i].astype(jnp.float32)
                      + b1 * es_ref[1, i].astype(jnp.float32))


def kernel(h, mps, mp_edge, gcn_W, gcn_bfc, gcn_bias, gcn_a, att_W, att_b,
           att_v):
    del mp_edge
    gwt = jnp.swapaxes(gcn_W, 1, 2)
    awt = jnp.swapaxes(att_W, 1, 2)
    gp = jnp.stack(
        [gcn_bfc, gcn_bias, jnp.broadcast_to(gcn_a[:, None], (4, HID))],
        axis=1)
    ab = att_b[:, None, :]
    av = att_v[:, None, :]

    def zmap(p, c, i):
        return (p * NB + jnp.where(c < 2, 0, i), 0)

    z = pl.pallas_call(
        _body,
        grid=(2, 3, NB),
        in_specs=[
            pl.BlockSpec((N, HID), lambda p, c, i: (p, 0)),
        ] + [
            pl.BlockSpec(
                (1, BM // 2, N),
                lambda p, c, i, q=q: (
                    2 * p + jnp.minimum(c, 1),
                    2 * jnp.where(c < 2, i, NB - 1) + q, 0))
            for q in range(2)
        ] + [
            pl.BlockSpec((1, HID, HID),
                         lambda p, c, i: (2 * p + jnp.minimum(c, 1), 0, 0)),
            pl.BlockSpec((1, 3, HID),
                         lambda p, c, i: (2 * p + jnp.minimum(c, 1), 0, 0)),
            pl.BlockSpec((1, HID, HID), lambda p, c, i: (p, 0, 0)),
            pl.BlockSpec((1, 1, HID), lambda p, c, i: (p, 0, 0)),
            pl.BlockSpec((1, 1, HID), lambda p, c, i: (p, 0, 0)),
        ],
        out_specs=pl.BlockSpec((BM, HID), zmap),
        out_shape=jax.ShapeDtypeStruct((2 * N, HID), jnp.float32),
        scratch_shapes=[
            pltpu.VMEM((N, HID), jnp.bfloat16),
            pltpu.VMEM((2, NB, BM, HID), jnp.bfloat16),
            pltpu.VMEM((2, 1, HID), jnp.float32),
        ],
        compiler_params=pltpu.CompilerParams(
            dimension_semantics=("parallel", "arbitrary", "arbitrary")),
    )(h, mps, mps, gwt, gp, awt, ab, av)
    return z


# speedup vs baseline: 1.1394x; 1.0492x over previous
import jax
import jax.numpy as jnp
from jax.experimental import pallas as pl
from jax.experimental.pallas import tpu as pltpu

HID = 256
N = 4096
BM = 1024
NB = N // BM
HB = BM // 2
RING = 4
HTOT = 8 * NB


def _adj_copy(adj_ref, abuf_ref, sem_ref, ht):
    kk = ht // (2 * NB)
    r = (ht % (2 * NB)) * HB
    slot = ht % RING
    return pltpu.make_async_copy(
        adj_ref.at[kk, pl.ds(r, HB), :],
        abuf_ref.at[slot],
        sem_ref.at[slot])


def _mix(cs_ref, av_ref, es_ref, z_ref, pair, i):
    c0, c1 = 2 * pair, 2 * pair + 1
    av = av_ref[pair, 0, :]
    l0 = jnp.sum(av * cs_ref[c0, 0, :]) * (1.0 / N)
    l1 = jnp.sum(av * cs_ref[c1, 0, :]) * (1.0 / N)
    m = jnp.maximum(l0, l1)
    x0 = jnp.exp(l0 - m)
    x1 = jnp.exp(l1 - m)
    b0 = x0 / (x0 + x1)
    b1 = x1 / (x0 + x1)
    z_ref[...] = (b0 * es_ref[c0, i].astype(jnp.float32)
                  + b1 * es_ref[c1, i].astype(jnp.float32))


def _body(h_ref, adj_ref, wt_ref, gp_ref, awt_ref, ab_ref, av_ref,
          z_ref, sf_ref, es_ref, cs_ref, abuf_ref, sem_ref):
    k = pl.program_id(0)
    i = pl.program_id(1)

    @pl.when(k < 4)
    def _():
        t = k * NB + i

        @pl.when(t == 0)
        def _():
            for j in range(4):
                _adj_copy(adj_ref, abuf_ref, sem_ref, jnp.int32(j)).start()

        @pl.when(t > 0)
        def _():
            for d in (2, 3):
                ht = 2 * t + d

                @pl.when(ht < HTOT)
                def _(ht=ht):
                    _adj_copy(adj_ref, abuf_ref, sem_ref, ht).start()

        @pl.when(i == 0)
        def _():
            sf_ref[...] = (
                jnp.dot(h_ref[...].astype(jnp.bfloat16),
                        wt_ref[0].astype(jnp.bfloat16),
                        preferred_element_type=jnp.float32)
                + gp_ref[0, 0, :][None, :]).astype(jnp.bfloat16)

        _adj_copy(adj_ref, abuf_ref, sem_ref, 2 * t).wait()
        _adj_copy(adj_ref, abuf_ref, sem_ref, 2 * t + 1).wait()

        bias = gp_ref[0, 1, :][None, :]
        a = gp_ref[0, 2, :][None, :]
        ot = jnp.dot(abuf_ref[(2 * t) % RING].astype(jnp.bfloat16),
                     sf_ref[...],
                     preferred_element_type=jnp.float32) + bias
        ob = jnp.dot(abuf_ref[(2 * t + 1) % RING].astype(jnp.bfloat16),
                     sf_ref[...],
                     preferred_element_type=jnp.float32) + bias
        et = jnp.where(ot > 0, ot, a * ot).astype(jnp.bfloat16)
        eb = jnp.where(ob > 0, ob, a * ob).astype(jnp.bfloat16)
        es_ref[k, i, :BM // 2] = et
        es_ref[k, i, BM // 2:] = eb
        awtb = awt_ref[0].astype(jnp.bfloat16)
        abr = ab_ref[k // 2, 0, :][None, :]
        t = (jnp.tanh(jnp.dot(et, awtb, preferred_element_type=jnp.float32)
                      + abr)
             + jnp.tanh(jnp.dot(eb, awtb, preferred_element_type=jnp.float32)
                        + abr))
        part = jnp.sum(t, axis=0, keepdims=True)

        @pl.when(i == 0)
        def _():
            cs_ref[k] = part

        @pl.when(i > 0)
        def _():
            cs_ref[k] = cs_ref[k] + part

    @pl.when(k == 2)
    def _():
        _mix(cs_ref, av_ref, es_ref, z_ref, 0, i)

    @pl.when(k == 4)
    def _():
        _mix(cs_ref, av_ref, es_ref, z_ref, 1, i)


def kernel(h, mps, mp_edge, gcn_W, gcn_bfc, gcn_bias, gcn_a, att_W, att_b,
           att_v):
    del mp_edge
    gwt = jnp.swapaxes(gcn_W, 1, 2)
    awt = jnp.swapaxes(att_W, 1, 2)
    gp = jnp.stack(
        [gcn_bfc, gcn_bias, jnp.broadcast_to(gcn_a[:, None], (4, HID))],
        axis=1)
    ab = att_b[:, None, :]
    av = att_v[:, None, :]

    def zmap(k, i):
        blk = jnp.where(k < 2, 0,
                        jnp.where(k == 2, i,
                                  jnp.where(k == 3, NB - 1, NB + i)))
        return (blk, 0)

    z = pl.pallas_call(
        _body,
        grid=(5, NB),
        in_specs=[
            pl.BlockSpec((N, HID), lambda k, i: (jnp.minimum(k, 3) // 2, 0)),
            pl.BlockSpec(memory_space=pl.ANY),
        ] + [
            pl.BlockSpec((1, HID, HID), lambda k, i: (jnp.minimum(k, 3), 0, 0)),
            pl.BlockSpec((1, 3, HID), lambda k, i: (jnp.minimum(k, 3), 0, 0)),
            pl.BlockSpec((1, HID, HID),
                         lambda k, i: (jnp.minimum(k, 3) // 2, 0, 0)),
            pl.BlockSpec((2, 1, HID), lambda k, i: (0, 0, 0)),
            pl.BlockSpec((2, 1, HID), lambda k, i: (0, 0, 0)),
        ],
        out_specs=pl.BlockSpec((BM, HID), zmap),
        out_shape=jax.ShapeDtypeStruct((2 * N, HID), jnp.float32),
        scratch_shapes=[
            pltpu.VMEM((N, HID), jnp.bfloat16),
            pltpu.VMEM((4, NB, BM, HID), jnp.bfloat16),
            pltpu.VMEM((4, 1, HID), jnp.float32),
            pltpu.VMEM((RING, HB, N), jnp.float32),
            pltpu.SemaphoreType.DMA((RING,)),
        ],
    )(h, mps, gwt, gp, awt, ab, av)
    return z
